# Initial kernel scaffold; baseline (speedup 1.0000x reference)
import jax
import jax.numpy as jnp
from jax.experimental import pallas as pl


def kernel(pred, target, masses, batch_index):
    raise NotImplementedError("write your pallas kernel here")


# speedup vs baseline: 6.7508x; 6.7508x over previous
import functools

import jax
import jax.numpy as jnp
from jax import lax
from jax.experimental import pallas as pl
from jax.experimental.pallas import tpu as pltpu
import jax.experimental.pallas.tpu_sc as plsc

NC = 2
NS = 16
NW = NC * NS
LANES = 16
GSEG = 1024
A_LEN = 4 * GSEG
TAIL = 128
ROW = A_LEN + TAIL
CHUNK = 2000


def _build_sc(n):
    np_w = n // NW
    nch = np_w // CHUNK
    assert np_w * NW == n and nch * CHUNK == np_w

    mesh = plsc.VectorSubcoreMesh(
        core_axis_name="c", subcore_axis_name="s", num_cores=NC,
        num_subcores=NS)

    @functools.partial(
        pl.kernel,
        out_type=jax.ShapeDtypeStruct((NW, ROW), jnp.float32),
        mesh=mesh,
        compiler_params=pltpu.CompilerParams(needs_layout_passes=False),
        scratch_types=[
            pltpu.VMEM((CHUNK * 6,), jnp.float32),
            pltpu.VMEM((CHUNK * 6,), jnp.float32),
            pltpu.VMEM((CHUNK,), jnp.float32),
            pltpu.VMEM((CHUNK,), jnp.int32),
            pltpu.VMEM((A_LEN,), jnp.float32),
            pltpu.VMEM((TAIL,), jnp.float32),
        ],
    )
    def sc_kernel(pred_hbm, target_hbm, masses_hbm, bidx_hbm, out_hbm,
                  pred_v, targ_v, mass_v, bidx_v, acc_ref, tail_ref):
        wid = lax.axis_index("s") * NC + lax.axis_index("c")
        base_w = wid * np_w

        zeros = jnp.zeros((LANES,), jnp.float32)

        def zero_body(k, carry):
            acc_ref[pl.ds(k * LANES, LANES)] = zeros
            return carry

        lax.fori_loop(0, A_LEN // LANES, zero_body, 0)

        iota6 = 6 * lax.iota(jnp.int32, LANES)

        def group_body(i, carry):
            accp, accv, accm, gmx = carry
            p0 = i * LANES
            base6 = 6 * p0 + iota6
            m = mass_v[pl.ds(p0, LANES)]
            g = bidx_v[pl.ds(p0, LANES)]
            p = [plsc.load_gather(pred_v, [base6 + c]) for c in range(6)]
            t = [plsc.load_gather(targ_v, [base6 + c]) for c in range(6)]
            d = [p[c] - t[c] for c in range(6)]
            accp = accp + d[0] * d[0] + d[1] * d[1] + d[2] * d[2]
            accv = accv + d[3] * d[3] + d[4] * d[4] + d[5] * d[5]
            accm = accm + m
            gmx = jnp.maximum(gmx, g)
            s4g = 4 * g
            plsc.addupdate_scatter(acc_ref, [s4g], m * d[3])
            plsc.addupdate_scatter(acc_ref, [s4g + 1], m * d[4])
            plsc.addupdate_scatter(acc_ref, [s4g + 2], m * d[5])
            ke = (0.5 * m) * (d[3] * (p[3] + t[3]) + d[4] * (p[4] + t[4])
                              + d[5] * (p[5] + t[5]))
            plsc.addupdate_scatter(acc_ref, [s4g + 3], ke)
            return (accp, accv, accm, gmx)

        def chunk_body(k, carry):
            base = base_w + k * CHUNK
            pltpu.sync_copy(pred_hbm.at[pl.ds(base * 6, CHUNK * 6)], pred_v)
            pltpu.sync_copy(target_hbm.at[pl.ds(base * 6, CHUNK * 6)], targ_v)
            pltpu.sync_copy(masses_hbm.at[pl.ds(base, CHUNK)], mass_v)
            pltpu.sync_copy(bidx_hbm.at[pl.ds(base, CHUNK)], bidx_v)
            return lax.fori_loop(0, CHUNK // LANES, group_body, carry)

        init = (zeros, zeros, zeros, jnp.full((LANES,), -1, jnp.int32))
        accp, accv, accm, gmx = lax.fori_loop(0, nch, chunk_body, init)

        tail_ref[pl.ds(0, LANES)] = accp
        tail_ref[pl.ds(LANES, LANES)] = accv
        tail_ref[pl.ds(2 * LANES, LANES)] = accm
        tail_ref[pl.ds(3 * LANES, LANES)] = gmx.astype(jnp.float32)
        for k in range(4, TAIL // LANES):
            tail_ref[pl.ds(k * LANES, LANES)] = zeros

        pltpu.sync_copy(acc_ref, out_hbm.at[wid, pl.ds(0, A_LEN)])
        pltpu.sync_copy(tail_ref, out_hbm.at[wid, pl.ds(A_LEN, TAIL)])

    return sc_kernel


def _build_tc(n):
    nf = float(n)

    def tc_body(part_ref, out_ref):
        x = part_ref[...]
        srow = jnp.sum(x, axis=0, keepdims=True)
        mrow = jnp.max(x, axis=0, keepdims=True)
        col = lax.broadcasted_iota(jnp.int32, (1, ROW), 1)
        in_a = col < A_LEN
        c4 = col % 4
        sq = srow * srow
        mom_sq = jnp.sum(jnp.where(in_a & (c4 < 3), sq, 0.0))
        ke_sq = jnp.sum(jnp.where(in_a & (c4 == 3), sq, 0.0))

        def tail_sum(slot, row):
            m = (col >= A_LEN + slot * LANES) & (col < A_LEN + (slot + 1) * LANES)
            return jnp.sum(jnp.where(m, row, 0.0))

        pos_sum = tail_sum(0, srow)
        vel_sum = tail_sum(1, srow)
        mass_sum = tail_sum(2, srow)
        gmax_m = (col >= A_LEN + 3 * LANES) & (col < A_LEN + 4 * LANES)
        gmax = jnp.max(jnp.where(gmax_m, mrow, -1.0))

        n_graphs = gmax + 1.0
        mass_scale = mass_sum / nf
        s_eff = jnp.where(mass_scale > 0.0, mass_scale, 1.0)
        inv2 = 1.0 / (s_eff * s_eff)

        pos_loss = pos_sum / (3.0 * nf)
        vel_loss = vel_sum / (3.0 * nf)
        momentum_loss = mom_sq * inv2 / (n_graphs * 3.0)
        energy_loss = ke_sq * inv2 / n_graphs
        total = (pos_loss + vel_loss + 0.1 * energy_loss
                 + 0.1 * momentum_loss)

        lane = lax.broadcasted_iota(jnp.int32, (1, 128), 1)
        out = jnp.where(lane == 0, total,
              jnp.where(lane == 1, pos_loss,
              jnp.where(lane == 2, vel_loss,
              jnp.where(lane == 3, energy_loss,
              jnp.where(lane == 4, momentum_loss, 0.0)))))
        out_ref[...] = out

    return pl.pallas_call(
        tc_body,
        out_shape=jax.ShapeDtypeStruct((1, 128), jnp.float32),
    )


@jax.jit
def kernel(pred, target, masses, batch_index):
    n = pred.shape[0]
    partials = _build_sc(n)(pred.reshape(-1), target.reshape(-1), masses,
                            batch_index.astype(jnp.int32))
    out = _build_tc(n)(partials)
    return (out[0, 0], out[0, 1], out[0, 2], out[0, 3], out[0, 4])
